# baseline (speedup 1.0000x reference)
import functools

import numpy as np
import jax
import jax.numpy as jnp
from jax import lax
from jax.experimental import pallas as pl
from jax.experimental.pallas import tpu as pltpu
from jax.experimental.pallas import tpu_sc as plsc

N_DECODER, N_SCENE, N_AGENT, N_PRED, N_STEP = 2, 16, 64, 6, 80
K_PRED = N_DECODER * N_PRED
MPA_NMS_THRESH = (2.5, 1.0, 2.0)

_NC, _NS, _NL = 2, 16, 16
_APW = 32
_NG = _APW // _NL

_RP = N_AGENT * N_PRED
_TC2 = 2 * N_STEP
_WT = N_AGENT * K_PRED * 2
_WY = N_AGENT * K_PRED


def _np_selectors():
    k = np.arange(_TC2)[None, :]
    t = np.arange(N_STEP)[:, None]
    s_e = (k == 2 * t).astype(np.float32)
    s_o = (k == 2 * t + 1).astype(np.float32)
    m = np.arange(_WT)
    a, u = m // 24, m % 24
    d, pc = u // 12, u % 12
    p, c = pc // 2, pc % 2
    src = (d * 2 + c) * _RP + a * N_PRED + p
    p_t = np.zeros((_WT, _WT), np.float32)
    p_t[src, m] = 1.0
    my = np.arange(_WY)
    ay, uy = my // K_PRED, my % K_PRED
    dy, py = uy // N_PRED, uy % N_PRED
    srcy = dy * _RP + ay * N_PRED + py
    p_y = np.zeros((_WY, _WY), np.float32)
    p_y[srcy, my] = 1.0
    ident = np.eye(N_STEP, dtype=np.float32)
    return s_e, s_o, p_t, p_y, ident


_S_E, _S_O, _P_T, _P_Y, _I80 = _np_selectors()
_DN_LL = (((1,), (1,)), ((), ()))
_DN_LR = (((1,), (0,)), ((), ()))


def _tc_trans_body(pos_ref, yaw_ref, se_ref, so_ref, i80_ref, pt_ref, py_ref,
                   otr_ref, oyw_ref):
    se, so = se_ref[...], so_ref[...]
    planes = []
    for d in range(N_DECODER):
        x = pos_ref[d, 0]
        planes.append(lax.dot_general(se, x, _DN_LL,
                                      preferred_element_type=jnp.float32,
                                      precision=lax.Precision.HIGHEST))
        planes.append(lax.dot_general(so, x, _DN_LL,
                                      preferred_element_type=jnp.float32,
                                      precision=lax.Precision.HIGHEST))
    c_all = jnp.concatenate(planes, axis=1)
    otr_ref[0] = lax.dot_general(c_all, pt_ref[...], _DN_LR,
                                 preferred_element_type=jnp.float32,
                                      precision=lax.Precision.HIGHEST)

    i80 = i80_ref[...]
    ys = []
    for d in range(N_DECODER):
        yv = yaw_ref[d, 0]
        ys.append(lax.dot_general(i80, yv, _DN_LL,
                                  preferred_element_type=jnp.float32,
                                      precision=lax.Precision.HIGHEST))
    cy = jnp.concatenate(ys, axis=1)
    oyw_ref[0] = lax.dot_general(cy, py_ref[...], _DN_LR,
                                 preferred_element_type=jnp.float32,
                                      precision=lax.Precision.HIGHEST)


def _tc_dist_body(pos_ref, reft_ref, owd_ref):
    rt = reft_ref[0]
    thresh = (rt[:, 0:1] * MPA_NMS_THRESH[0]
              + rt[:, 1:2] * MPA_NMS_THRESH[1]
              + rt[:, 2:3] * MPA_NMS_THRESH[2])
    v = pos_ref[:, 0]
    lane_i = jax.lax.broadcasted_iota(jnp.int32, (N_AGENT, _TC2), 1)
    even_mask = jnp.where(lane_i % 2 == 0, 1.0, 0.0)
    sl = [v[k // N_PRED, :, (k % N_PRED) * 160:(k % N_PRED + 1) * 160]
          for k in range(K_PRED)]
    cols = []
    for k1 in range(K_PRED):
        for k2 in range(k1 + 1, K_PRED):
            dd = sl[k1] - sl[k2]
            d2 = dd * dd
            s2 = d2 + jnp.roll(d2, -1, axis=1)
            dist = jnp.sqrt(s2) * even_mask
            md = jnp.sum(dist, axis=-1, keepdims=True) * (1.0 / N_STEP)
            cols.append(jnp.where(md < thresh, 1.0, 0.0))
    zero = jnp.zeros((N_AGENT, 1), jnp.float32)
    grid_cols = [[None] * K_PRED for _ in range(K_PRED)]
    ci = 0
    for k1 in range(K_PRED):
        grid_cols[k1][k1] = zero
        for k2 in range(k1 + 1, K_PRED):
            grid_cols[k1][k2] = cols[ci]
            grid_cols[k2][k1] = cols[ci]
            ci += 1
    owd_ref[0] = jnp.concatenate([c for row in grid_cols for c in row], axis=-1)


def _tc_main_body(pos_ref, posr_ref, yaw_ref, reft_ref, se_ref, so_ref,
                  i80_ref, pt_ref, py_ref, otr_ref, oyw_ref, owd_ref):
    _tc_trans_body(posr_ref, yaw_ref, se_ref, so_ref, i80_ref, pt_ref, py_ref,
                   otr_ref, oyw_ref)
    _tc_dist_body(pos_ref, reft_ref, owd_ref)


def _sc_body(conf_hbm, wd_hbm, out_hbm, conf_v, wd_v, out_v):
    wid = lax.axis_index("s") * _NC + lax.axis_index("c")
    s_idx = wid // 2
    a0 = (wid % 2) * _APW

    for d in range(N_DECODER):
        pltpu.sync_copy(conf_hbm.at[d, s_idx, pl.ds(a0, _APW), :], conf_v.at[d])
    pltpu.sync_copy(wd_hbm.at[s_idx, pl.ds(a0, _APW)], wd_v)

    for g in range(_NG):
        lane_a = jax.lax.iota(jnp.int32, _NL) + g * _NL

        sraw = []
        for dd in range(N_DECODER):
            fd = jnp.full((_NL,), dd, jnp.int32)
            rows = [plsc.load_gather(
                        conf_v, [fd, lane_a, jnp.full((_NL,), p, jnp.int32)])
                    for p in range(N_PRED)]
            m = rows[0]
            for r_ in rows[1:]:
                m = jnp.maximum(m, r_)
            es = [jnp.exp(r_ - m) for r_ in rows]
            z = es[0]
            for e_ in es[1:]:
                z = z + e_
            sraw.extend([e_ / z for e_ in es])
        tot = sraw[0]
        for v_ in sraw[1:]:
            tot = tot + v_
        sc = [v_ / tot for v_ in sraw]

        rank = []
        for i in range(K_PRED):
            r_acc = jnp.zeros((_NL,), jnp.int32)
            for j in range(K_PRED):
                if j < i:
                    c = sc[j] >= sc[i]
                elif j > i:
                    c = sc[j] > sc[i]
                else:
                    continue
                r_acc = r_acc + c.astype(jnp.int32)
            rank.append(r_acc)

        s = list(sc)
        for r in range(K_PRED):
            k_vec = jnp.zeros((_NL,), jnp.int32)
            s_k = jnp.zeros((_NL,), jnp.float32)
            for i in range(K_PRED):
                oh = rank[i] == r
                k_vec = jnp.where(oh, jnp.int32(i), k_vec)
                s_k = jnp.where(oh, s[i], s_k)
            col = k_vec * K_PRED
            anym = (k_vec < 0)
            for j in range(K_PRED):
                wdj = plsc.load_gather(wd_v, [lane_a, col + j])
                anym = anym | ((wdj > 0.5) & (s[j] > s_k))
            for i in range(K_PRED):
                s[i] = jnp.where(anym & (k_vec == i), jnp.float32(0.001), s[i])

        tot2 = s[0]
        for v_ in s[1:]:
            tot2 = tot2 + v_
        for i in range(K_PRED):
            plsc.store_scatter(out_v, [lane_a, jnp.full((_NL,), i, jnp.int32)],
                               s[i] / tot2)

    pltpu.sync_copy(out_v, out_hbm.at[s_idx, pl.ds(a0, _APW)])


def _nms_scores_sc(pred_conf, wd3):
    mesh = plsc.VectorSubcoreMesh(core_axis_name="c", subcore_axis_name="s",
                                  num_cores=_NC, num_subcores=_NS)
    f = functools.partial(
        pl.kernel,
        mesh=mesh,
        compiler_params=pltpu.CompilerParams(needs_layout_passes=False),
        out_type=jax.ShapeDtypeStruct((N_SCENE, N_AGENT, K_PRED), jnp.float32),
        scratch_types=[
            pltpu.VMEM((N_DECODER, _APW, N_PRED), jnp.float32),
            pltpu.VMEM((_APW, K_PRED * K_PRED), jnp.float32),
            pltpu.VMEM((_APW, K_PRED), jnp.float32),
        ],
    )(_sc_body)
    return f(pred_conf, wd3)


def kernel(pred_valid, pred_conf, pred_pos, pred_yaw_bbox, ref_type):
    S, A, K, T = N_SCENE, N_AGENT, K_PRED, N_STEP
    pos_flat = pred_pos.reshape(N_DECODER, S, A, N_PRED * T * 2)
    pos_rows = pred_pos.reshape(N_DECODER, S, _RP, _TC2)
    yaw_rows = pred_yaw_bbox.reshape(N_DECODER, S, _RP, T)
    s_e = jnp.asarray(_S_E)
    s_o = jnp.asarray(_S_O)
    i80 = jnp.asarray(_I80)
    p_t = jnp.asarray(_P_T)
    p_y = jnp.asarray(_P_Y)

    trajs_flat, yaw_out_flat, wd3 = pl.pallas_call(
        _tc_main_body,
        grid=(S,),
        in_specs=[
            pl.BlockSpec((N_DECODER, 1, A, N_PRED * T * 2), lambda s: (0, s, 0, 0)),
            pl.BlockSpec((N_DECODER, 1, _RP, _TC2), lambda s: (0, s, 0, 0)),
            pl.BlockSpec((N_DECODER, 1, _RP, T), lambda s: (0, s, 0, 0)),
            pl.BlockSpec((1, A, 3), lambda s: (s, 0, 0)),
            pl.BlockSpec((N_STEP, _TC2), lambda s: (0, 0)),
            pl.BlockSpec((N_STEP, _TC2), lambda s: (0, 0)),
            pl.BlockSpec((N_STEP, N_STEP), lambda s: (0, 0)),
            pl.BlockSpec((_WT, _WT), lambda s: (0, 0)),
            pl.BlockSpec((_WY, _WY), lambda s: (0, 0)),
        ],
        out_specs=[
            pl.BlockSpec((1, T, _WT), lambda s: (s, 0, 0)),
            pl.BlockSpec((1, T, _WY), lambda s: (s, 0, 0)),
            pl.BlockSpec((1, A, K * K), lambda s: (s, 0, 0)),
        ],
        out_shape=[
            jax.ShapeDtypeStruct((S, T, _WT), jnp.float32),
            jax.ShapeDtypeStruct((S, T, _WY), jnp.float32),
            jax.ShapeDtypeStruct((S, A, K * K), jnp.float32),
        ],
    )(pos_flat, pos_rows, yaw_rows, ref_type, s_e, s_o, i80, p_t, p_y)

    scores = _nms_scores_sc(pred_conf, wd3)

    waymo_valid = jnp.broadcast_to(pred_valid[:, None, :], (S, T, A))
    waymo_trajs = trajs_flat.reshape(S, T, A, K, 2)
    waymo_yaw = yaw_out_flat.reshape(S, T, A, K, 1)
    return (waymo_valid, waymo_trajs, scores, waymo_yaw)


# speedup vs baseline: 1.0535x; 1.0535x over previous
import functools

import numpy as np
import jax
import jax.numpy as jnp
from jax import lax
from jax.experimental import pallas as pl
from jax.experimental.pallas import tpu as pltpu
from jax.experimental.pallas import tpu_sc as plsc

N_DECODER, N_SCENE, N_AGENT, N_PRED, N_STEP = 2, 16, 64, 6, 80
K_PRED = N_DECODER * N_PRED
MPA_NMS_THRESH = (2.5, 1.0, 2.0)

_NC, _NS, _NL = 2, 16, 16
_APW = 32
_NG = _APW // _NL

_RP = N_AGENT * N_PRED
_TC2 = 2 * N_STEP
_WT = N_AGENT * K_PRED * 2
_WY = N_AGENT * K_PRED


def _np_selectors():
    k = np.arange(_TC2)[None, :]
    t = np.arange(N_STEP)[:, None]
    s_e = (k == 2 * t).astype(np.float32)
    s_o = (k == 2 * t + 1).astype(np.float32)
    m = np.arange(_WT)
    a, u = m // 24, m % 24
    d, pc = u // 12, u % 12
    p, c = pc // 2, pc % 2
    src = (d * 2 + c) * _RP + a * N_PRED + p
    p_t = np.zeros((_WT, _WT), np.float32)
    p_t[src, m] = 1.0
    my = np.arange(_WY)
    ay, uy = my // K_PRED, my % K_PRED
    dy, py = uy // N_PRED, uy % N_PRED
    srcy = dy * _RP + ay * N_PRED + py
    p_y = np.zeros((_WY, _WY), np.float32)
    p_y[srcy, my] = 1.0
    ident = np.eye(N_STEP, dtype=np.float32)
    return s_e, s_o, p_t, p_y, ident


_S_E, _S_O, _P_T, _P_Y, _I80 = _np_selectors()
_DN_LL = (((1,), (1,)), ((), ()))
_DN_LR = (((1,), (0,)), ((), ()))


def _tc_trans_body(pos_ref, yaw_ref, se_ref, so_ref, i80_ref, pt_ref, py_ref,
                   otr_ref, oyw_ref):
    se, so = se_ref[...], so_ref[...]
    planes = []
    for d in range(N_DECODER):
        x = pos_ref[d, 0]
        planes.append(lax.dot_general(se, x, _DN_LL,
                                      preferred_element_type=jnp.float32))
        planes.append(lax.dot_general(so, x, _DN_LL,
                                      preferred_element_type=jnp.float32))
    c_all = jnp.concatenate(planes, axis=1)
    otr_ref[0] = lax.dot_general(c_all, pt_ref[...], _DN_LR,
                                 preferred_element_type=jnp.float32)

    i80 = i80_ref[...]
    ys = []
    for d in range(N_DECODER):
        yv = yaw_ref[d, 0]
        ys.append(lax.dot_general(i80, yv, _DN_LL,
                                  preferred_element_type=jnp.float32))
    cy = jnp.concatenate(ys, axis=1)
    oyw_ref[0] = lax.dot_general(cy, py_ref[...], _DN_LR,
                                 preferred_element_type=jnp.float32)


def _tc_dist_body(pos_ref, reft_ref, owd_ref):
    rt = reft_ref[0]
    thresh = (rt[:, 0:1] * MPA_NMS_THRESH[0]
              + rt[:, 1:2] * MPA_NMS_THRESH[1]
              + rt[:, 2:3] * MPA_NMS_THRESH[2])
    v = pos_ref[:, 0]
    lane_i = jax.lax.broadcasted_iota(jnp.int32, (N_AGENT, _TC2), 1)
    even_mask = jnp.where(lane_i % 2 == 0, 1.0, 0.0)
    sl = [v[k // N_PRED, :, (k % N_PRED) * 160:(k % N_PRED + 1) * 160]
          for k in range(K_PRED)]
    cols = []
    for k1 in range(K_PRED):
        for k2 in range(k1 + 1, K_PRED):
            dd = sl[k1] - sl[k2]
            d2 = dd * dd
            s2 = d2 + jnp.roll(d2, -1, axis=1)
            dist = jnp.sqrt(s2) * even_mask
            md = jnp.sum(dist, axis=-1, keepdims=True) * (1.0 / N_STEP)
            cols.append(jnp.where(md < thresh, 1.0, 0.0))
    zero = jnp.zeros((N_AGENT, 1), jnp.float32)
    grid_cols = [[None] * K_PRED for _ in range(K_PRED)]
    ci = 0
    for k1 in range(K_PRED):
        grid_cols[k1][k1] = zero
        for k2 in range(k1 + 1, K_PRED):
            grid_cols[k1][k2] = cols[ci]
            grid_cols[k2][k1] = cols[ci]
            ci += 1
    owd_ref[0] = jnp.concatenate([c for row in grid_cols for c in row], axis=-1)


def _tc_main_body(pos_ref, posr_ref, yaw_ref, reft_ref, se_ref, so_ref,
                  i80_ref, pt_ref, py_ref, otr_ref, oyw_ref, owd_ref):
    _tc_trans_body(posr_ref, yaw_ref, se_ref, so_ref, i80_ref, pt_ref, py_ref,
                   otr_ref, oyw_ref)
    _tc_dist_body(pos_ref, reft_ref, owd_ref)


def _sc_body(conf_hbm, wd_hbm, out_hbm, conf_v, wd_v, out_v):
    wid = lax.axis_index("s") * _NC + lax.axis_index("c")
    s_idx = wid // 2
    a0 = (wid % 2) * _APW

    for d in range(N_DECODER):
        pltpu.sync_copy(conf_hbm.at[d, s_idx, pl.ds(a0, _APW), :], conf_v.at[d])
    pltpu.sync_copy(wd_hbm.at[s_idx, pl.ds(a0, _APW)], wd_v)

    for g in range(_NG):
        lane_a = jax.lax.iota(jnp.int32, _NL) + g * _NL

        sraw = []
        for dd in range(N_DECODER):
            fd = jnp.full((_NL,), dd, jnp.int32)
            rows = [plsc.load_gather(
                        conf_v, [fd, lane_a, jnp.full((_NL,), p, jnp.int32)])
                    for p in range(N_PRED)]
            m = rows[0]
            for r_ in rows[1:]:
                m = jnp.maximum(m, r_)
            es = [jnp.exp(r_ - m) for r_ in rows]
            z = es[0]
            for e_ in es[1:]:
                z = z + e_
            sraw.extend([e_ / z for e_ in es])
        tot = sraw[0]
        for v_ in sraw[1:]:
            tot = tot + v_
        sc = [v_ / tot for v_ in sraw]

        rank = []
        for i in range(K_PRED):
            r_acc = jnp.zeros((_NL,), jnp.int32)
            for j in range(K_PRED):
                if j < i:
                    c = sc[j] >= sc[i]
                elif j > i:
                    c = sc[j] > sc[i]
                else:
                    continue
                r_acc = r_acc + c.astype(jnp.int32)
            rank.append(r_acc)

        s = list(sc)
        for r in range(K_PRED):
            k_vec = jnp.zeros((_NL,), jnp.int32)
            s_k = jnp.zeros((_NL,), jnp.float32)
            for i in range(K_PRED):
                oh = rank[i] == r
                k_vec = jnp.where(oh, jnp.int32(i), k_vec)
                s_k = jnp.where(oh, s[i], s_k)
            col = k_vec * K_PRED
            anym = (k_vec < 0)
            for j in range(K_PRED):
                wdj = plsc.load_gather(wd_v, [lane_a, col + j])
                anym = anym | ((wdj > 0.5) & (s[j] > s_k))
            for i in range(K_PRED):
                s[i] = jnp.where(anym & (k_vec == i), jnp.float32(0.001), s[i])

        tot2 = s[0]
        for v_ in s[1:]:
            tot2 = tot2 + v_
        for i in range(K_PRED):
            plsc.store_scatter(out_v, [lane_a, jnp.full((_NL,), i, jnp.int32)],
                               s[i] / tot2)

    pltpu.sync_copy(out_v, out_hbm.at[s_idx, pl.ds(a0, _APW)])


def _nms_scores_sc(pred_conf, wd3):
    mesh = plsc.VectorSubcoreMesh(core_axis_name="c", subcore_axis_name="s",
                                  num_cores=_NC, num_subcores=_NS)
    f = functools.partial(
        pl.kernel,
        mesh=mesh,
        compiler_params=pltpu.CompilerParams(needs_layout_passes=False),
        out_type=jax.ShapeDtypeStruct((N_SCENE, N_AGENT, K_PRED), jnp.float32),
        scratch_types=[
            pltpu.VMEM((N_DECODER, _APW, N_PRED), jnp.float32),
            pltpu.VMEM((_APW, K_PRED * K_PRED), jnp.float32),
            pltpu.VMEM((_APW, K_PRED), jnp.float32),
        ],
    )(_sc_body)
    return f(pred_conf, wd3)


def kernel(pred_valid, pred_conf, pred_pos, pred_yaw_bbox, ref_type):
    S, A, K, T = N_SCENE, N_AGENT, K_PRED, N_STEP
    pos_flat = pred_pos.reshape(N_DECODER, S, A, N_PRED * T * 2)
    pos_rows = pred_pos.reshape(N_DECODER, S, _RP, _TC2)
    yaw_rows = pred_yaw_bbox.reshape(N_DECODER, S, _RP, T)
    s_e = jnp.asarray(_S_E)
    s_o = jnp.asarray(_S_O)
    i80 = jnp.asarray(_I80)
    p_t = jnp.asarray(_P_T)
    p_y = jnp.asarray(_P_Y)

    trajs_flat, yaw_out_flat, wd3 = pl.pallas_call(
        _tc_main_body,
        grid=(S,),
        in_specs=[
            pl.BlockSpec((N_DECODER, 1, A, N_PRED * T * 2), lambda s: (0, s, 0, 0)),
            pl.BlockSpec((N_DECODER, 1, _RP, _TC2), lambda s: (0, s, 0, 0)),
            pl.BlockSpec((N_DECODER, 1, _RP, T), lambda s: (0, s, 0, 0)),
            pl.BlockSpec((1, A, 3), lambda s: (s, 0, 0)),
            pl.BlockSpec((N_STEP, _TC2), lambda s: (0, 0)),
            pl.BlockSpec((N_STEP, _TC2), lambda s: (0, 0)),
            pl.BlockSpec((N_STEP, N_STEP), lambda s: (0, 0)),
            pl.BlockSpec((_WT, _WT), lambda s: (0, 0)),
            pl.BlockSpec((_WY, _WY), lambda s: (0, 0)),
        ],
        out_specs=[
            pl.BlockSpec((1, T, _WT), lambda s: (s, 0, 0)),
            pl.BlockSpec((1, T, _WY), lambda s: (s, 0, 0)),
            pl.BlockSpec((1, A, K * K), lambda s: (s, 0, 0)),
        ],
        out_shape=[
            jax.ShapeDtypeStruct((S, T, _WT), jnp.float32),
            jax.ShapeDtypeStruct((S, T, _WY), jnp.float32),
            jax.ShapeDtypeStruct((S, A, K * K), jnp.float32),
        ],
    )(pos_flat, pos_rows, yaw_rows, ref_type, s_e, s_o, i80, p_t, p_y)

    scores = _nms_scores_sc(pred_conf, wd3)

    waymo_valid = jnp.broadcast_to(pred_valid[:, None, :], (S, T, A))
    waymo_trajs = trajs_flat.reshape(S, T, A, K, 2)
    waymo_yaw = yaw_out_flat.reshape(S, T, A, K, 1)
    return (waymo_valid, waymo_trajs, scores, waymo_yaw)
